# baseline (speedup 1.0000x reference)
import functools

import jax
import jax.numpy as jnp
from jax.experimental import pallas as pl
from jax.experimental.pallas import tpu as pltpu

_F32 = jnp.float32


def _leaky(v):
    return jnp.where(v >= 0, v, 0.2 * v)


def _sr(a):
    z = jnp.zeros((a.shape[0], 1), _F32)
    return jnp.concatenate([z, a[:, :-1]], axis=1)


def _sl(a):
    z = jnp.zeros((a.shape[0], 1), _F32)
    return jnp.concatenate([a[:, 1:], z], axis=1)


def _body(P_ref, W10_ref, W11_ref, W12_ref, W13_ref, be1_ref, Wc2_ref, be2_ref,
          Wfi_ref, bfi_ref, Wfo_ref, bfo_ref,
          E1_ref, E2_ref, E3_ref, E4_ref, T1_ref, T2_ref, T3_ref, T4_ref,
          D1e_ref, D1o_ref, bd1_ref, D2a_ref, D2b_ref, bd2_ref,
          y_ref, loss_ref, ppl_ref,
          esq_ref, hist_ref, lacc_ref, *, n_batch, n_codes, n_tok):
    b = pl.program_id(0)
    E_refs = (E1_ref, E2_ref, E3_ref, E4_ref)
    T_refs = (T1_ref, T2_ref, T3_ref, T4_ref)

    @pl.when(b == 0)
    def _init():
        for j, Er in enumerate(E_refs):
            e = Er[...]
            esq_ref[:, j:j + 1] = jnp.sum(e * e, axis=1, keepdims=True)
        hist_ref[...] = jnp.zeros_like(hist_ref)
        lacc_ref[...] = jnp.zeros_like(lacc_ref)

    dot = functools.partial(jnp.dot, preferred_element_type=_F32)

    p0 = P_ref[0, 0]
    p1 = P_ref[0, 1]
    p2 = P_ref[0, 2]
    p3 = P_ref[0, 3]
    rhs0 = jnp.concatenate([_sr(p3), p1], axis=1)
    rhs1 = jnp.concatenate([p0, p2], axis=1)
    rhs2 = jnp.concatenate([p1, p3], axis=1)
    rhs3 = jnp.concatenate([p2, _sl(p0)], axis=1)
    h1 = (dot(W10_ref[...], rhs0) + dot(W11_ref[...], rhs1)
          + dot(W12_ref[...], rhs2) + dot(W13_ref[...], rhs3))
    h1 = _leaky(h1 + be1_ref[...])
    h1e = h1[:, :128]
    h1o = h1[:, 128:]

    r2 = jnp.concatenate([_sr(h1o), h1e, h1o, _sl(h1e)], axis=0)
    h = _leaky(dot(Wc2_ref[...], r2) + be2_ref[...])

    zp = dot(Wfi_ref[...], h) + bfi_ref[...]
    zh = jnp.round(zp)
    Wfo = Wfo_ref[...]
    zfsq = bfo_ref[...] * jnp.ones((1, n_tok), _F32)
    for dch in range(4):
        zfsq = zfsq + Wfo[:, dch:dch + 1] * zh[dch:dch + 1, :]
    r = h - zfsq

    iota = jax.lax.broadcasted_iota(jnp.int32, (n_codes, n_tok), 0)
    lsum = jnp.zeros((1, 1), _F32)
    for j in range(4):
        scores = dot(E_refs[j][...], r)
        rsq = jnp.sum(r * r, axis=0, keepdims=True)
        d = (rsq + esq_ref[:, j:j + 1]) - 2.0 * scores
        minv = jnp.min(d, axis=0, keepdims=True)
        idx = jnp.min(jnp.where(d == minv, iota, n_codes), axis=0,
                      keepdims=True)
        enc = (iota == idx).astype(_F32)
        q = dot(T_refs[j][...], enc)
        r = r - q
        lsum = lsum + jnp.sum(
            jnp.sum(r * r, axis=0, keepdims=True), axis=1, keepdims=True)
        hist_ref[:, j:j + 1] += jnp.sum(enc, axis=1, keepdims=True)
    lacc_ref[...] += lsum
    z = h - r

    y1e = _leaky(dot(D1e_ref[...], jnp.concatenate([z, _sr(z)], axis=0))
                 + bd1_ref[...])
    y1o = _leaky(dot(D1o_ref[...], jnp.concatenate([z, _sl(z)], axis=0))
                 + bd1_ref[...])

    rhsA = jnp.concatenate(
        [jnp.concatenate([y1e, y1o], axis=1),
         jnp.concatenate([_sr(y1o), y1e], axis=1)], axis=0)
    pA = dot(D2a_ref[...], rhsA) + bd2_ref[...]
    rhsB = jnp.concatenate(
        [jnp.concatenate([y1o, _sl(y1e)], axis=1),
         jnp.concatenate([y1e, y1o], axis=1)], axis=0)
    pB = dot(D2b_ref[...], rhsB) + bd2_ref[...]
    y_ref[0, 0] = pA[:, :128]
    y_ref[0, 1] = pB[:, :128]
    y_ref[0, 2] = pA[:, 128:]
    y_ref[0, 3] = pB[:, 128:]

    @pl.when(b == n_batch - 1)
    def _fin():
        n_rows = jnp.float32(n_batch * n_tok)
        loss_ref[...] = lacc_ref[...] * (0.25 / (n_rows * 512.0))
        avg = hist_ref[...] * (1.0 / n_rows)
        ent = jnp.sum(avg * jnp.log(avg + 1e-10), axis=0, keepdims=True)
        ppl_ref[...] = jnp.sum(jnp.exp(-ent), axis=1, keepdims=True) * 0.25


def kernel(x, We1, be1, We2, be2, Wfi, bfi, Wfo, bfo,
           E1, E2, E3, E4, Wd1, bd1, Wd2, bd2):
    B, C, T = x.shape
    H = We1.shape[0]
    NE = E1.shape[0]
    TT = T // 4

    P = jnp.stack([x[:, :, k::4] for k in range(4)], axis=1)

    W1k = [We1[:, :, k] for k in range(4)]
    Wc2 = jnp.concatenate([We2[:, :, k] for k in range(4)], axis=1)
    D1e = jnp.concatenate([Wd1[:, :, 1].T, Wd1[:, :, 3].T], axis=1)
    D1o = jnp.concatenate([Wd1[:, :, 2].T, Wd1[:, :, 0].T], axis=1)
    D2a = jnp.concatenate([Wd2[:, :, 1].T, Wd2[:, :, 3].T], axis=1)
    D2b = jnp.concatenate([Wd2[:, :, 0].T, Wd2[:, :, 2].T], axis=1)
    col = lambda v: v.reshape(-1, 1)

    full = lambda a: pl.BlockSpec(a.shape, lambda b: (0,) * a.ndim)
    ins = [
        P, W1k[0], W1k[1], W1k[2], W1k[3], col(be1), Wc2, col(be2),
        Wfi, col(bfi), Wfo, col(bfo),
        E1, E2, E3, E4, E1.T, E2.T, E3.T, E4.T,
        D1e, D1o, col(bd1), D2a, D2b, col(bd2),
    ]
    in_specs = [pl.BlockSpec((1, 4, C, TT), lambda b: (b, 0, 0, 0))]
    in_specs += [full(a) for a in ins[1:]]

    out_shapes = (
        jax.ShapeDtypeStruct((B, 4, C, TT), _F32),
        jax.ShapeDtypeStruct((1, 1), _F32),
        jax.ShapeDtypeStruct((1, 1), _F32),
    )
    out_specs = (
        pl.BlockSpec((1, 4, C, TT), lambda b: (b, 0, 0, 0)),
        pl.BlockSpec((1, 1), lambda b: (0, 0)),
        pl.BlockSpec((1, 1), lambda b: (0, 0)),
    )

    y4, loss, ppl = pl.pallas_call(
        functools.partial(_body, n_batch=B, n_codes=NE, n_tok=TT),
        grid=(B,),
        in_specs=in_specs,
        out_specs=out_specs,
        out_shape=out_shapes,
        scratch_shapes=[
            pltpu.VMEM((NE, 4), _F32),
            pltpu.VMEM((NE, 4), _F32),
            pltpu.VMEM((1, 1), _F32),
        ],
        compiler_params=pltpu.CompilerParams(
            dimension_semantics=("arbitrary",)),
    )(*ins)

    y = jnp.transpose(y4, (0, 2, 3, 1)).reshape(B, C, T)
    return (y, loss[0, 0], ppl[0, 0])


# speedup vs baseline: 2.5088x; 2.5088x over previous
import functools

import jax
import jax.numpy as jnp
from jax.experimental import pallas as pl
from jax.experimental.pallas import tpu as pltpu

_F32 = jnp.float32


def _leaky(v):
    return jnp.where(v >= 0, v, 0.2 * v)


def _sr(a):
    z = jnp.zeros((a.shape[0], 1), _F32)
    return jnp.concatenate([z, a[:, :-1]], axis=1)


def _sl(a):
    z = jnp.zeros((a.shape[0], 1), _F32)
    return jnp.concatenate([a[:, 1:], z], axis=1)


def _iota2(shape, dim):
    return jax.lax.broadcasted_iota(jnp.int32, shape, dim)


def _body(x_ref, W10_ref, W11_ref, W12_ref, W13_ref, be1_ref, Wc2_ref, be2_ref,
          Wfi_ref, bfi_ref, Wfo_ref, bfo_ref,
          E1_ref, E2_ref, E3_ref, E4_ref, T1_ref, T2_ref, T3_ref, T4_ref,
          D1e_ref, D1o_ref, bd1_ref, D2a_ref, D2b_ref, bd2_ref,
          y_ref, loss_ref, ppl_ref,
          esq_ref, hist_ref, lacc_ref, se_ref, so_ref, se2_ref, so2_ref,
          ra_ref, rb_ref, *, n_batch, n_codes, n_tok):
    b = pl.program_id(0)
    E_refs = (E1_ref, E2_ref, E3_ref, E4_ref)
    T_refs = (T1_ref, T2_ref, T3_ref, T4_ref)

    @pl.when(b == 0)
    def _init():
        for j, Er in enumerate(E_refs):
            e = Er[...]
            esq_ref[:, j:j + 1] = jnp.sum(e * e, axis=1, keepdims=True)
        hist_ref[...] = jnp.zeros_like(hist_ref)
        lacc_ref[...] = jnp.zeros_like(lacc_ref)
        se_ref[...] = (_iota2((512, 256), 0) == 2 * _iota2((512, 256), 1)
                       ).astype(_F32)
        so_ref[...] = (_iota2((512, 256), 0) == 2 * _iota2((512, 256), 1) + 1
                       ).astype(_F32)
        se2_ref[...] = (_iota2((256, 128), 0) == 2 * _iota2((256, 128), 1)
                        ).astype(_F32)
        so2_ref[...] = (_iota2((256, 128), 0) == 2 * _iota2((256, 128), 1) + 1
                        ).astype(_F32)
        r256 = _iota2((256, 512), 0)
        c512 = _iota2((256, 512), 1)
        ra_ref[...] = (jnp.where(r256 < 128, 4 * r256, 4 * (r256 - 128) + 2)
                       == c512).astype(_F32)
        rb_ref[...] = (jnp.where(r256 < 128, 4 * r256 + 1, 4 * (r256 - 128) + 3)
                       == c512).astype(_F32)

    dot = functools.partial(jnp.dot, preferred_element_type=_F32)

    xb = x_ref[0]
    xe = dot(xb, se_ref[...])
    xo = dot(xb, so_ref[...])

    h1 = (dot(W10_ref[...], _sr(xo)) + dot(W11_ref[...], xe)
          + dot(W12_ref[...], xo) + dot(W13_ref[...], _sl(xe)))
    h1 = _leaky(h1 + be1_ref[...])
    h1e = dot(h1, se2_ref[...])
    h1o = dot(h1, so2_ref[...])

    r2 = jnp.concatenate([_sr(h1o), h1e, h1o, _sl(h1e)], axis=0)
    h = _leaky(dot(Wc2_ref[...], r2) + be2_ref[...])

    zp = dot(Wfi_ref[...], h) + bfi_ref[...]
    zh = jnp.round(zp)
    Wfo = Wfo_ref[...]
    zfsq = bfo_ref[...] * jnp.ones((1, n_tok), _F32)
    for dch in range(4):
        zfsq = zfsq + Wfo[:, dch:dch + 1] * zh[dch:dch + 1, :]
    r = h - zfsq

    iota = _iota2((n_codes, n_tok), 0)
    lsum = jnp.zeros((1, 1), _F32)
    for j in range(4):
        scores = dot(E_refs[j][...], r)
        rsq = jnp.sum(r * r, axis=0, keepdims=True)
        d = (rsq + esq_ref[:, j:j + 1]) - 2.0 * scores
        minv = jnp.min(d, axis=0, keepdims=True)
        idx = jnp.min(jnp.where(d == minv, iota, n_codes), axis=0,
                      keepdims=True)
        enc = (iota == idx).astype(_F32)
        q = dot(T_refs[j][...], enc)
        r = r - q
        lsum = lsum + jnp.sum(
            jnp.sum(r * r, axis=0, keepdims=True), axis=1, keepdims=True)
        hist_ref[:, j:j + 1] += jnp.sum(enc, axis=1, keepdims=True)
    lacc_ref[...] += lsum
    z = h - r

    y1e = _leaky(dot(D1e_ref[...], jnp.concatenate([z, _sr(z)], axis=0))
                 + bd1_ref[...])
    y1o = _leaky(dot(D1o_ref[...], jnp.concatenate([z, _sl(z)], axis=0))
                 + bd1_ref[...])

    rhsA = jnp.concatenate(
        [jnp.concatenate([y1e, y1o], axis=1),
         jnp.concatenate([_sr(y1o), y1e], axis=1)], axis=0)
    pA = dot(D2a_ref[...], rhsA) + bd2_ref[...]
    rhsB = jnp.concatenate(
        [jnp.concatenate([y1o, _sl(y1e)], axis=1),
         jnp.concatenate([y1e, y1o], axis=1)], axis=0)
    pB = dot(D2b_ref[...], rhsB) + bd2_ref[...]
    y_ref[0] = dot(pA, ra_ref[...]) + dot(pB, rb_ref[...])

    @pl.when(b == n_batch - 1)
    def _fin():
        n_rows = jnp.float32(n_batch * n_tok)
        loss_ref[...] = lacc_ref[...] * (0.25 / (n_rows * 512.0))
        avg = hist_ref[...] * (1.0 / n_rows)
        ent = jnp.sum(avg * jnp.log(avg + 1e-10), axis=0, keepdims=True)
        ppl_ref[...] = jnp.sum(jnp.exp(-ent), axis=1, keepdims=True) * 0.25


def kernel(x, We1, be1, We2, be2, Wfi, bfi, Wfo, bfo,
           E1, E2, E3, E4, Wd1, bd1, Wd2, bd2):
    B, C, T = x.shape
    NE = E1.shape[0]
    TT = T // 4

    W1k = [We1[:, :, k] for k in range(4)]
    Wc2 = jnp.concatenate([We2[:, :, k] for k in range(4)], axis=1)
    D1e = jnp.concatenate([Wd1[:, :, 1].T, Wd1[:, :, 3].T], axis=1)
    D1o = jnp.concatenate([Wd1[:, :, 2].T, Wd1[:, :, 0].T], axis=1)
    D2a = jnp.concatenate([Wd2[:, :, 1].T, Wd2[:, :, 3].T], axis=1)
    D2b = jnp.concatenate([Wd2[:, :, 0].T, Wd2[:, :, 2].T], axis=1)
    col = lambda v: v.reshape(-1, 1)

    full = lambda a: pl.BlockSpec(a.shape, lambda b: (0,) * a.ndim)
    ins = [
        x, W1k[0], W1k[1], W1k[2], W1k[3], col(be1), Wc2, col(be2),
        Wfi, col(bfi), Wfo, col(bfo),
        E1, E2, E3, E4, E1.T, E2.T, E3.T, E4.T,
        D1e, D1o, col(bd1), D2a, D2b, col(bd2),
    ]
    in_specs = [pl.BlockSpec((1, C, T), lambda b: (b, 0, 0))]
    in_specs += [full(a) for a in ins[1:]]

    out_shapes = (
        jax.ShapeDtypeStruct((B, C, T), _F32),
        jax.ShapeDtypeStruct((1, 1), _F32),
        jax.ShapeDtypeStruct((1, 1), _F32),
    )
    out_specs = (
        pl.BlockSpec((1, C, T), lambda b: (b, 0, 0)),
        pl.BlockSpec((1, 1), lambda b: (0, 0)),
        pl.BlockSpec((1, 1), lambda b: (0, 0)),
    )

    y, loss, ppl = pl.pallas_call(
        functools.partial(_body, n_batch=B, n_codes=NE, n_tok=TT),
        grid=(B,),
        in_specs=in_specs,
        out_specs=out_specs,
        out_shape=out_shapes,
        scratch_shapes=[
            pltpu.VMEM((NE, 4), _F32),
            pltpu.VMEM((NE, 4), _F32),
            pltpu.VMEM((1, 1), _F32),
            pltpu.VMEM((512, 256), _F32),
            pltpu.VMEM((512, 256), _F32),
            pltpu.VMEM((256, 128), _F32),
            pltpu.VMEM((256, 128), _F32),
            pltpu.VMEM((256, 512), _F32),
            pltpu.VMEM((256, 512), _F32),
        ],
        compiler_params=pltpu.CompilerParams(
            dimension_semantics=("arbitrary",)),
    )(*ins)

    return (y, loss[0, 0], ppl[0, 0])


# speedup vs baseline: 5.5456x; 2.2105x over previous
import functools

import jax
import jax.numpy as jnp
from jax.experimental import pallas as pl
from jax.experimental.pallas import tpu as pltpu

_F32 = jnp.float32


def _leaky(v):
    return jnp.where(v >= 0, v, 0.2 * v)


def _sr(a):
    z = jnp.zeros((a.shape[0], 1), _F32)
    return jnp.concatenate([z, a[:, :-1]], axis=1)


def _sl(a):
    z = jnp.zeros((a.shape[0], 1), _F32)
    return jnp.concatenate([a[:, 1:], z], axis=1)


def _iota2(shape, dim):
    return jax.lax.broadcasted_iota(jnp.int32, shape, dim)


def _body(x_ref, W10_ref, W11_ref, W12_ref, W13_ref, be1_ref, Wc2_ref, be2_ref,
          Wfi_ref, bfi_ref, Wfo_ref, bfo_ref,
          E1_ref, E2_ref, E3_ref, E4_ref, T1_ref, T2_ref, T3_ref, T4_ref,
          D1e_ref, D1o_ref, bd1_ref, D2a_ref, D2b_ref, bd2_ref,
          y_ref, loss_ref, ppl_ref,
          esq_ref, hist_ref, lacc_ref, se_ref, so_ref, se2_ref, so2_ref,
          ra_ref, rb_ref, *, n_steps, bb, n_codes, tt):
    b = pl.program_id(0)
    E_refs = (E1_ref, E2_ref, E3_ref, E4_ref)
    T_refs = (T1_ref, T2_ref, T3_ref, T4_ref)
    ntok = bb * tt

    @pl.when(b == 0)
    def _init():
        for j, Er in enumerate(E_refs):
            e = Er[...]
            esq_ref[:, j:j + 1] = jnp.sum(e * e, axis=1, keepdims=True)
        hist_ref[...] = jnp.zeros_like(hist_ref)
        lacc_ref[...] = jnp.zeros_like(lacc_ref)
        se_ref[...] = (_iota2((512, 256), 0) == 2 * _iota2((512, 256), 1)
                       ).astype(_F32)
        so_ref[...] = (_iota2((512, 256), 0) == 2 * _iota2((512, 256), 1) + 1
                       ).astype(_F32)
        se2_ref[...] = (_iota2((256, 128), 0) == 2 * _iota2((256, 128), 1)
                        ).astype(_F32)
        so2_ref[...] = (_iota2((256, 128), 0) == 2 * _iota2((256, 128), 1) + 1
                        ).astype(_F32)
        r256 = _iota2((256, 512), 0)
        c512 = _iota2((256, 512), 1)
        ra_ref[...] = (jnp.where(r256 < 128, 4 * r256, 4 * (r256 - 128) + 2)
                       == c512).astype(_F32)
        rb_ref[...] = (jnp.where(r256 < 128, 4 * r256 + 1, 4 * (r256 - 128) + 3)
                       == c512).astype(_F32)

    dot = functools.partial(jnp.dot, preferred_element_type=_F32)
    cat = jnp.concatenate

    se, so = se_ref[...], so_ref[...]
    sro, e_, o_, sle = [], [], [], []
    for i in range(bb):
        xb = x_ref[i]
        xe = dot(xb, se)
        xo = dot(xb, so)
        sro.append(_sr(xo))
        e_.append(xe)
        o_.append(xo)
        sle.append(_sl(xe))
    h1 = (dot(W10_ref[...], cat(sro, axis=1)) + dot(W11_ref[...], cat(e_, axis=1))
          + dot(W12_ref[...], cat(o_, axis=1)) + dot(W13_ref[...], cat(sle, axis=1)))
    h1 = _leaky(h1 + be1_ref[...])

    se2, so2 = se2_ref[...], so2_ref[...]
    r2c = []
    for i in range(bb):
        h1b = h1[:, 256 * i:256 * (i + 1)]
        h1e = dot(h1b, se2)
        h1o = dot(h1b, so2)
        r2c.append(cat([_sr(h1o), h1e, h1o, _sl(h1e)], axis=0))
    h = _leaky(dot(Wc2_ref[...], cat(r2c, axis=1)) + be2_ref[...])

    zp = dot(Wfi_ref[...], h) + bfi_ref[...]
    zh = jnp.round(zp)
    Wfo = Wfo_ref[...]
    zfsq = bfo_ref[...] * jnp.ones((1, ntok), _F32)
    for dch in range(4):
        zfsq = zfsq + Wfo[:, dch:dch + 1] * zh[dch:dch + 1, :]
    r = h - zfsq

    iota = _iota2((n_codes, ntok), 0)
    lsum = jnp.zeros((1, 1), _F32)
    for j in range(4):
        scores = dot(E_refs[j][...], r)
        rsq = jnp.sum(r * r, axis=0, keepdims=True)
        d = (rsq + esq_ref[:, j:j + 1]) - 2.0 * scores
        minv = jnp.min(d, axis=0, keepdims=True)
        idx = jnp.min(jnp.where(d == minv, iota, n_codes), axis=0,
                      keepdims=True)
        enc = (iota == idx).astype(_F32)
        q = dot(T_refs[j][...], enc)
        r = r - q
        lsum = lsum + jnp.sum(
            jnp.sum(r * r, axis=0, keepdims=True), axis=1, keepdims=True)
        hist_ref[:, j:j + 1] += jnp.sum(enc, axis=1, keepdims=True)
    lacc_ref[...] += lsum
    z = h - r

    zc, zsr, zsl = [], [], []
    for i in range(bb):
        zb = z[:, tt * i:tt * (i + 1)]
        zc.append(zb)
        zsr.append(_sr(zb))
        zsl.append(_sl(zb))
    y1e = _leaky(dot(D1e_ref[...], cat([cat(zc, axis=1), cat(zsr, axis=1)],
                                       axis=0)) + bd1_ref[...])
    y1o = _leaky(dot(D1o_ref[...], cat([cat(zc, axis=1), cat(zsl, axis=1)],
                                       axis=0)) + bd1_ref[...])

    rA, rB = [], []
    for i in range(bb):
        ye = y1e[:, tt * i:tt * (i + 1)]
        yo = y1o[:, tt * i:tt * (i + 1)]
        rA.append(cat([cat([ye, yo], axis=1), cat([_sr(yo), ye], axis=1)],
                      axis=0))
        rB.append(cat([cat([yo, _sl(ye)], axis=1), cat([ye, yo], axis=1)],
                      axis=0))
    pA = dot(D2a_ref[...], cat(rA, axis=1)) + bd2_ref[...]
    pB = dot(D2b_ref[...], cat(rB, axis=1)) + bd2_ref[...]
    ra, rb_ = ra_ref[...], rb_ref[...]
    for i in range(bb):
        y_ref[i] = (dot(pA[:, 256 * i:256 * (i + 1)], ra)
                    + dot(pB[:, 256 * i:256 * (i + 1)], rb_))

    @pl.when(b == n_steps - 1)
    def _fin():
        n_rows = jnp.float32(n_steps * ntok)
        loss_ref[...] = lacc_ref[...] * (0.25 / (n_rows * 512.0))
        avg = hist_ref[...] * (1.0 / n_rows)
        ent = jnp.sum(avg * jnp.log(avg + 1e-10), axis=0, keepdims=True)
        ppl_ref[...] = jnp.sum(jnp.exp(-ent), axis=1, keepdims=True) * 0.25


def kernel(x, We1, be1, We2, be2, Wfi, bfi, Wfo, bfo,
           E1, E2, E3, E4, Wd1, bd1, Wd2, bd2):
    B, C, T = x.shape
    NE = E1.shape[0]
    TT = T // 4
    BB = 4 if B % 4 == 0 else (2 if B % 2 == 0 else 1)
    n_steps = B // BB

    W1k = [We1[:, :, k] for k in range(4)]
    Wc2 = jnp.concatenate([We2[:, :, k] for k in range(4)], axis=1)
    D1e = jnp.concatenate([Wd1[:, :, 1].T, Wd1[:, :, 3].T], axis=1)
    D1o = jnp.concatenate([Wd1[:, :, 2].T, Wd1[:, :, 0].T], axis=1)
    D2a = jnp.concatenate([Wd2[:, :, 1].T, Wd2[:, :, 3].T], axis=1)
    D2b = jnp.concatenate([Wd2[:, :, 0].T, Wd2[:, :, 2].T], axis=1)
    col = lambda v: v.reshape(-1, 1)

    full = lambda a: pl.BlockSpec(a.shape, lambda b: (0,) * a.ndim)
    ins = [
        x, W1k[0], W1k[1], W1k[2], W1k[3], col(be1), Wc2, col(be2),
        Wfi, col(bfi), Wfo, col(bfo),
        E1, E2, E3, E4, E1.T, E2.T, E3.T, E4.T,
        D1e, D1o, col(bd1), D2a, D2b, col(bd2),
    ]
    in_specs = [pl.BlockSpec((BB, C, T), lambda b: (b, 0, 0))]
    in_specs += [full(a) for a in ins[1:]]

    out_shapes = (
        jax.ShapeDtypeStruct((B, C, T), _F32),
        jax.ShapeDtypeStruct((1, 1), _F32),
        jax.ShapeDtypeStruct((1, 1), _F32),
    )
    out_specs = (
        pl.BlockSpec((BB, C, T), lambda b: (b, 0, 0)),
        pl.BlockSpec((1, 1), lambda b: (0, 0)),
        pl.BlockSpec((1, 1), lambda b: (0, 0)),
    )

    y, loss, ppl = pl.pallas_call(
        functools.partial(_body, n_steps=n_steps, bb=BB, n_codes=NE, tt=TT),
        grid=(n_steps,),
        in_specs=in_specs,
        out_specs=out_specs,
        out_shape=out_shapes,
        scratch_shapes=[
            pltpu.VMEM((NE, 4), _F32),
            pltpu.VMEM((NE, 4), _F32),
            pltpu.VMEM((1, 1), _F32),
            pltpu.VMEM((512, 256), _F32),
            pltpu.VMEM((512, 256), _F32),
            pltpu.VMEM((256, 128), _F32),
            pltpu.VMEM((256, 128), _F32),
            pltpu.VMEM((256, 512), _F32),
            pltpu.VMEM((256, 512), _F32),
        ],
        compiler_params=pltpu.CompilerParams(
            dimension_semantics=("arbitrary",)),
    )(*ins)

    return (y, loss[0, 0], ppl[0, 0])
